# Initial kernel scaffold; baseline (speedup 1.0000x reference)
import jax
import jax.numpy as jnp
from jax.experimental import pallas as pl


def kernel(x, embedding):
    raise NotImplementedError("write your pallas kernel here")


# speedup vs baseline: 1.7495x; 1.7495x over previous
import functools

import jax
import jax.numpy as jnp
from jax import lax
from jax.experimental import pallas as pl
from jax.experimental.pallas import tpu as pltpu
from jax.experimental.pallas import tpu_sc as plsc

_DIM = 64
_NC = 2
_NS = 16
_NW = _NC * _NS
_CHUNK = 128


@functools.lru_cache(maxsize=None)
def _build(n_chunks: int):
    mesh = plsc.VectorSubcoreMesh(
        core_axis_name="c", subcore_axis_name="s", num_cores=_NC, num_subcores=_NS
    )

    @functools.partial(
        pl.kernel,
        out_type=jax.ShapeDtypeStruct((_NW, n_chunks, _CHUNK, _DIM), jnp.float32),
        mesh=mesh,
        compiler_params=pltpu.CompilerParams(use_tc_tiling_on_sc=False),
        scratch_types=[
            pltpu.VMEM((n_chunks, _CHUNK), jnp.int32),
            pltpu.VMEM((_CHUNK, _DIM), jnp.float32),
            pltpu.VMEM((_CHUNK, _DIM), jnp.float32),
            pltpu.SemaphoreType.DMA,
            pltpu.SemaphoreType.DMA,
            pltpu.SemaphoreType.DMA,
            pltpu.SemaphoreType.DMA,
        ],
    )
    def emb(table, idx, out, idx_v, rows0, rows1, gsem0, gsem1, osem0, osem1):
        wid = lax.axis_index("s") * _NC + lax.axis_index("c")
        rows = (rows0, rows1)
        gsem = (gsem0, gsem1)
        osem = (osem0, osem1)

        pltpu.sync_copy(idx.at[wid], idx_v)

        def gstart(g, b):
            pltpu.async_copy(table.at[idx_v.at[g]], rows[b], gsem[b])

        def gwait(b):
            pltpu.make_async_copy(table.at[idx_v.at[0]], rows[b], gsem[b]).wait()

        def ostart(g, b):
            pltpu.async_copy(rows[b], out.at[wid, g], osem[b])

        def owait(b):
            pltpu.make_async_copy(rows[b], out.at[wid, 0], osem[b]).wait()

        gstart(0, 0)
        gwait(0)
        ostart(0, 0)
        gstart(1, 1)

        def steady(i, _):
            g = 1 + i * 2
            for j in range(2):
                b = (1 + j) % 2
                gwait(b)
                ostart(g + j, b)
                owait(1 - b)
                gstart(g + j + 1, 1 - b)
            return 0

        lax.fori_loop(0, (n_chunks - 2) // 2, steady, 0, unroll=False)

        g_last = n_chunks - 1
        b_last = g_last % 2
        gwait(b_last)
        ostart(g_last, b_last)
        owait(0)
        owait(1)

    return emb


def kernel(x, embedding):
    b_total = x.size
    idx = x.reshape(-1).astype(jnp.int32)
    grain = 2 * _NW * _CHUNK
    b_pad = ((b_total + grain - 1) // grain) * grain
    if b_pad != b_total:
        idx = jnp.concatenate([idx, jnp.zeros((b_pad - b_total,), jnp.int32)])
    n_chunks = b_pad // (_NW * _CHUNK)
    idx = idx.reshape(_NW, n_chunks, _CHUNK)
    out = _build(n_chunks)(embedding, idx)
    out = out.reshape(b_pad, _DIM)
    if b_pad != b_total:
        out = out[:b_total]
    return out.reshape(x.shape + (_DIM,))


# speedup vs baseline: 1.8672x; 1.0673x over previous
import functools

import jax
import jax.numpy as jnp
from jax import lax
from jax.experimental import pallas as pl
from jax.experimental.pallas import tpu as pltpu
from jax.experimental.pallas import tpu_sc as plsc

_NC = 2
_NS = 16
_NW = _NC * _NS
_RCHUNK = 16


@functools.lru_cache(maxsize=None)
def _build(b_rows: int, p: int, dim: int):
    rows_per_w = b_rows // _NW
    n_chunks = rows_per_w // _RCHUNK
    mesh = plsc.VectorSubcoreMesh(
        core_axis_name="c", subcore_axis_name="s", num_cores=_NC, num_subcores=_NS
    )

    @functools.partial(
        pl.kernel,
        out_type=jax.ShapeDtypeStruct((b_rows, p, dim), jnp.float32),
        mesh=mesh,
        compiler_params=pltpu.CompilerParams(use_tc_tiling_on_sc=False),
        scratch_types=[
            pltpu.VMEM((rows_per_w, p), jnp.int32),
            pltpu.VMEM((_RCHUNK, p, dim), jnp.float32),
            pltpu.VMEM((_RCHUNK, p, dim), jnp.float32),
            pltpu.SemaphoreType.DMA,
            pltpu.SemaphoreType.DMA,
            pltpu.SemaphoreType.DMA,
            pltpu.SemaphoreType.DMA,
        ],
    )
    def emb(table, idx, out, idx_v, rows0, rows1, gsem0, gsem1, osem0, osem1):
        wid = lax.axis_index("s") * _NC + lax.axis_index("c")
        base = wid * rows_per_w
        rows = (rows0, rows1)
        gsem = (gsem0, gsem1)
        osem = (osem0, osem1)

        pltpu.sync_copy(idx.at[pl.ds(base, rows_per_w)], idx_v)

        def gstart(g, b):
            for k in range(_RCHUNK):
                pltpu.async_copy(
                    table.at[idx_v.at[g * _RCHUNK + k]],
                    rows[b].at[k],
                    gsem[b],
                )

        def gwait(b):
            pltpu.make_async_copy(
                out.at[pl.ds(base, _RCHUNK)], rows[b], gsem[b]
            ).wait()

        def ostart(g, b):
            pltpu.async_copy(
                rows[b], out.at[pl.ds(base + g * _RCHUNK, _RCHUNK)], osem[b]
            )

        def owait(b):
            pltpu.make_async_copy(
                rows[b], out.at[pl.ds(base, _RCHUNK)], osem[b]
            ).wait()

        gstart(0, 0)
        gwait(0)
        ostart(0, 0)
        gstart(1, 1)

        def steady(i, _):
            g = 1 + i * 2
            for j in range(2):
                b = (1 + j) % 2
                gwait(b)
                ostart(g + j, b)
                owait(1 - b)
                gstart(g + j + 1, 1 - b)
            return 0

        lax.fori_loop(0, (n_chunks - 2) // 2, steady, 0, unroll=False)

        g_last = n_chunks - 1
        b_last = g_last % 2
        gwait(b_last)
        ostart(g_last, b_last)
        owait(0)
        owait(1)

    return emb


def kernel(x, embedding):
    b_rows, p = x.shape
    dim = embedding.shape[1]
    idx = x.astype(jnp.int32)
    grain = 2 * _NW * _RCHUNK
    b_pad = ((b_rows + grain - 1) // grain) * grain
    if b_pad != b_rows:
        idx = jnp.concatenate(
            [idx, jnp.zeros((b_pad - b_rows, p), jnp.int32)], axis=0
        )
    out = _build(b_pad, p, dim)(embedding, idx)
    if b_pad != b_rows:
        out = out[:b_rows]
    return out
